# baseline (speedup 1.0000x reference)
import functools

import jax
import jax.numpy as jnp
from jax import lax
from jax.experimental import pallas as pl
from jax.experimental.pallas import tpu as pltpu
from jax.experimental.pallas import tpu_sc as plsc

NUM_ROWS = 1000000
PAD_COLS = 1000064
EMBED = 64
ROWP = 128
BATCH = 16384

NC = 2
NS = 16
L = 16
NW = NC * NS
B_PER_W = BATCH // NW
CU = 256
NCHUNKS = (PAD_COLS + CU - 1) // CU
TRIPS = (NCHUNKS + NW - 1) // NW
QUAD_TRIPS = (TRIPS + 3) // 4
U0_MAX = PAD_COLS - CU
SENT = 0x7FFFFFF


def _stage_lists(src_hbm, stage, lu, lc, t):
    iota = lax.iota(jnp.int32, L)

    def group(g, cnt):
        uv = stage[pl.ds(g * L, L)]
        m = ((uv >> 8) & (NW - 1)) == t
        cs = plsc.cumsum(m.astype(jnp.int32))
        pos = cnt + cs - 1
        b = g * L + iota
        pk = (uv & 255) | (b << 8)
        plsc.store_scatter(lu, [pos], pk, mask=m)
        plsc.store_scatter(lc, [pos], uv >> 8, mask=m)
        return cnt + cs[15]

    pltpu.sync_copy(src_hbm, stage.at[pl.ds(0, BATCH)])
    cnt = lax.fori_loop(0, BATCH // L, group, jnp.int32(0))
    plsc.store_scatter(lc, [cnt + iota], jnp.full((L,), SENT, jnp.int32))
    return cnt


def _process_chunk(c, cnt, state, tab_hbm, slab, sem_slab, lu, lc, chunkbuf,
                   rowbufs, dst_hbm, sem_row, fire_next_c):
    (fired,) = state
    iota = lax.iota(jnp.int32, L)
    u0 = pl.multiple_of(jnp.minimum(c * CU, U0_MAX), 128)
    adj = c * CU - u0

    def scan(g, k):
        cv = lc[pl.ds(g * L, L)]
        m = cv == c
        cs = plsc.cumsum(m.astype(jnp.int32))
        plsc.store_scatter(chunkbuf, [k + cs - 1], lu[pl.ds(g * L, L)],
                           mask=m)
        return k + cs[15]

    k = lax.fori_loop(0, (cnt + L - 1) // L, scan, jnp.int32(0))

    pltpu.make_async_copy(tab_hbm.at[:, pl.ds(0, CU)],
                          slab.at[:, pl.ds(0, CU)], sem_slab).wait()

    def hit(h, fired):
        pk = chunkbuf[pl.ds(h, L)][0]
        col = (pk & 255) + adj
        bh = pk >> 8
        slot = fired & 15
        colv = jnp.full((L,), col, jnp.int32)
        for q in range(EMBED // L):
            dq = q * L + iota
            vals = plsc.load_gather(slab, [dq, colv])
            rowbufs[pl.ds(slot * ROWP + q * L, L)] = vals
        pltpu.async_copy(rowbufs.at[pl.ds(slot * ROWP, ROWP)],
                         dst_hbm.at[bh], sem_row)
        fired = fired + 1

        @pl.when(fired >= 16)
        def _():
            pltpu.make_async_copy(dst_hbm.at[0], rowbufs.at[pl.ds(0, ROWP)],
                                  sem_row).wait()
        return fired

    fired = lax.fori_loop(0, k, hit, fired)

    u0n = pl.multiple_of(jnp.minimum(fire_next_c * CU, U0_MAX), 128)
    pltpu.async_copy(tab_hbm.at[:, pl.ds(u0n, CU)],
                     slab.at[:, pl.ds(0, CU)], sem_slab)
    return (fired,)


def _table_pass(t, src_hbm, dst_hbm, tab_hbm, stage, lu, lc, chunkbuf,
                slabs, rowbufs, sem_slabs, sem_row):
    cnt = _stage_lists(src_hbm, stage, lu, lc, t)
    for j in range(4):
        cj = t + NW * j
        u0j = pl.multiple_of(jnp.minimum(cj * CU, U0_MAX), 128)
        pltpu.async_copy(tab_hbm.at[:, pl.ds(u0j, CU)],
                         slabs[j].at[:, pl.ds(0, CU)], sem_slabs[j])

    def quad(i, state):
        for j in range(4):
            state = _process_chunk(
                t + NW * (4 * i + j), cnt, state, tab_hbm, slabs[j],
                sem_slabs[j], lu, lc, chunkbuf, rowbufs, dst_hbm,
                sem_row, t + NW * (4 * i + j + 4))
        return state

    (fired,) = lax.fori_loop(0, QUAD_TRIPS, quad, (jnp.int32(0),))

    for j in range(4):
        pltpu.make_async_copy(tab_hbm.at[:, pl.ds(0, CU)],
                              slabs[j].at[:, pl.ds(0, CU)], sem_slabs[j]).wait()

    def drain(h, carry):
        pltpu.make_async_copy(dst_hbm.at[0], rowbufs.at[pl.ds(0, ROWP)],
                              sem_row).wait()
        return carry

    lax.fori_loop(0, jnp.minimum(fired, 15), drain, 0)


def _streamer_body(user_hbm, item_hbm, tu_hbm, ti_hbm, a_hbm, b_hbm,
                   lu, lc, chunkbuf, slab0, slab1, slab2, slab3, rowbufs,
                   sem_s0, sem_s1, sem_s2, sem_s3, sem_row):
    t = lax.axis_index("s") * NC + lax.axis_index("c")
    slabs = [slab0, slab1, slab2, slab3]
    sem_slabs = [sem_s0, sem_s1, sem_s2, sem_s3]
    _table_pass(t, user_hbm, a_hbm, tu_hbm, chunkbuf, lu, lc, chunkbuf,
                slabs, rowbufs, sem_slabs, sem_row)
    _table_pass(t, item_hbm, b_hbm, ti_hbm, chunkbuf, lu, lc, chunkbuf,
                slabs, rowbufs, sem_slabs, sem_row)


def _finish_body(a_hbm, b_hbm, w_hbm, out_hbm, rows_a, rows_b, w_v, out_v, sem):
    t = lax.axis_index("s") * NC + lax.axis_index("c")
    base = t * B_PER_W
    ca = pltpu.async_copy(
        a_hbm.at[pl.ds(base, B_PER_W), pl.ds(0, EMBED)], rows_a, sem)
    cb = pltpu.async_copy(
        b_hbm.at[pl.ds(base, B_PER_W), pl.ds(0, EMBED)], rows_b, sem)
    pltpu.sync_copy(w_hbm, w_v)
    ca.wait()
    cb.wait()
    wq = [w_v[pl.ds(q * L, L)] for q in range(EMBED // L)]
    lane = lax.iota(jnp.int32, L)

    def elem(b, carry):
        ur = rows_a.at[b]
        ir = rows_b.at[b]
        s = ur[pl.ds(0, L)] * ir[pl.ds(0, L)] * wq[0]
        for q in range(1, EMBED // L):
            s = s + ur[pl.ds(q * L, L)] * ir[pl.ds(q * L, L)] * wq[q]
        tot = jnp.sum(s)
        plsc.store_scatter(out_v, [jnp.full((L,), b, jnp.int32)],
                           jnp.full((L,), 0.0, jnp.float32) + tot,
                           mask=lane < 1)
        return carry

    lax.fori_loop(0, B_PER_W, elem, 0, unroll=4)

    def group(g, carry):
        v = out_v[pl.ds(g * L, L)]
        out_v[pl.ds(g * L, L)] = 1.0 / (1.0 + jnp.exp(-v))
        return carry

    lax.fori_loop(0, B_PER_W // L, group, 0, unroll=4)
    pltpu.sync_copy(out_v, out_hbm.at[pl.ds(base, B_PER_W)])


def kernel(user, item, user_table, item_table, edge_weight):
    mesh = plsc.VectorSubcoreMesh(
        core_axis_name="c", subcore_axis_name="s", num_cores=NC, num_subcores=NS)
    streamer = functools.partial(
        pl.kernel,
        out_type=(jax.ShapeDtypeStruct((BATCH, ROWP), jnp.float32),
                  jax.ShapeDtypeStruct((BATCH, ROWP), jnp.float32)),
        mesh=mesh,
        compiler_params=pltpu.CompilerParams(
            needs_layout_passes=False, use_tc_tiling_on_sc=True),
        scratch_types=[
            pltpu.VMEM((BATCH + L,), jnp.int32),
            pltpu.VMEM((BATCH + L,), jnp.int32),
            pltpu.VMEM((BATCH + L,), jnp.int32),
            pltpu.VMEM((EMBED, CU), jnp.float32),
            pltpu.VMEM((EMBED, CU), jnp.float32),
            pltpu.VMEM((EMBED, CU), jnp.float32),
            pltpu.VMEM((EMBED, CU), jnp.float32),
            pltpu.VMEM((16 * ROWP,), jnp.float32),
            pltpu.SemaphoreType.DMA,
            pltpu.SemaphoreType.DMA,
            pltpu.SemaphoreType.DMA,
            pltpu.SemaphoreType.DMA,
            pltpu.SemaphoreType.DMA,
        ],
    )(_streamer_body)
    finisher = functools.partial(
        pl.kernel,
        out_type=jax.ShapeDtypeStruct((BATCH,), jnp.float32),
        mesh=mesh,
        compiler_params=pltpu.CompilerParams(
            needs_layout_passes=False, use_tc_tiling_on_sc=False),
        scratch_types=[
            pltpu.VMEM((B_PER_W, EMBED), jnp.float32),
            pltpu.VMEM((B_PER_W, EMBED), jnp.float32),
            pltpu.VMEM((EMBED,), jnp.float32),
            pltpu.VMEM((B_PER_W,), jnp.float32),
            pltpu.SemaphoreType.DMA,
        ],
    )(_finish_body)

    a, b = streamer(user.astype(jnp.int32), item.astype(jnp.int32),
                    user_table.T, item_table.T)
    return finisher(a, b, edge_weight.reshape(EMBED))


# speedup vs baseline: 1.0226x; 1.0226x over previous
import functools

import jax
import jax.numpy as jnp
from jax import lax
from jax.experimental import pallas as pl
from jax.experimental.pallas import tpu as pltpu
from jax.experimental.pallas import tpu_sc as plsc

NUM_ROWS = 1000000
PAD_COLS = 1000064
EMBED = 64
ROWP = 128
BATCH = 16384

NC = 2
NS = 16
L = 16
NW = NC * NS
B_PER_W = BATCH // NW
CU = 256
NCHUNKS = (PAD_COLS + CU - 1) // CU
TRIPS = (NCHUNKS + NW - 1) // NW
TRIPLE_TRIPS = TRIPS // 3
U0_MAX = PAD_COLS - CU
SENT = 0x7FFFFFF


def _stage_lists(src_hbm, stage, lu, lc, t):
    iota = lax.iota(jnp.int32, L)

    def group(g, cnt):
        uv = stage[pl.ds(g * L, L)]
        m = ((uv >> 8) & (NW - 1)) == t
        cs = plsc.cumsum(m.astype(jnp.int32))
        pos = cnt + cs - 1
        b = g * L + iota
        pk = (uv & 255) | (b << 8)
        plsc.store_scatter(lu, [pos], pk, mask=m)
        plsc.store_scatter(lc, [pos], uv >> 8, mask=m)
        return cnt + cs[15]

    pltpu.sync_copy(src_hbm, stage.at[pl.ds(0, BATCH)])
    cnt = lax.fori_loop(0, BATCH // L, group, jnp.int32(0))
    plsc.store_scatter(lc, [cnt + iota], jnp.full((L,), SENT, jnp.int32))
    return cnt


def _process_chunk(c, cnt, state, tab_hbm, slab, sem_slab, lu, lc, chunkbuf,
                   rowbufs, dst_hbm, sem_row, fire_next_c):
    (fired,) = state
    iota = lax.iota(jnp.int32, L)
    u0 = pl.multiple_of(jnp.minimum(c * CU, U0_MAX), 128)
    adj = c * CU - u0

    def scan(g, k):
        cv = lc[pl.ds(g * L, L)]
        m = cv == c
        cs = plsc.cumsum(m.astype(jnp.int32))
        plsc.store_scatter(chunkbuf, [k + cs - 1], lu[pl.ds(g * L, L)],
                           mask=m)
        return k + cs[15]

    k = lax.fori_loop(0, (cnt + L - 1) // L, scan, jnp.int32(0))

    pltpu.make_async_copy(tab_hbm.at[:, pl.ds(0, CU)],
                          slab.at[:, pl.ds(0, CU)], sem_slab).wait()

    def hit(h, fired):
        pk = chunkbuf[pl.ds(h, L)][0]
        col = (pk & 255) + adj
        bh = pk >> 8
        slot = fired & 15
        colv = jnp.full((L,), col, jnp.int32)
        for q in range(EMBED // L):
            dq = q * L + iota
            vals = plsc.load_gather(slab, [dq, colv])
            rowbufs[pl.ds(slot * ROWP + q * L, L)] = vals
        pltpu.async_copy(rowbufs.at[pl.ds(slot * ROWP, ROWP)],
                         dst_hbm.at[bh], sem_row)
        fired = fired + 1

        @pl.when(fired >= 16)
        def _():
            pltpu.make_async_copy(dst_hbm.at[0], rowbufs.at[pl.ds(0, ROWP)],
                                  sem_row).wait()
        return fired

    fired = lax.fori_loop(0, k, hit, fired)

    u0n = pl.multiple_of(jnp.minimum(fire_next_c * CU, U0_MAX), 128)
    pltpu.async_copy(tab_hbm.at[:, pl.ds(u0n, CU)],
                     slab.at[:, pl.ds(0, CU)], sem_slab)
    return (fired,)


def _table_pass(t, src_hbm, dst_hbm, tab_hbm, stage, lu, lc, chunkbuf,
                slabs, rowbufs, sem_slabs, sem_row):
    cnt = _stage_lists(src_hbm, stage, lu, lc, t)
    for j in range(3):
        cj = t + NW * j
        u0j = pl.multiple_of(jnp.minimum(cj * CU, U0_MAX), 128)
        pltpu.async_copy(tab_hbm.at[:, pl.ds(u0j, CU)],
                         slabs[j].at[:, pl.ds(0, CU)], sem_slabs[j])

    def triple(i, state):
        for j in range(3):
            state = _process_chunk(
                t + NW * (3 * i + j), cnt, state, tab_hbm, slabs[j],
                sem_slabs[j], lu, lc, chunkbuf, rowbufs, dst_hbm,
                sem_row, t + NW * (3 * i + j + 3))
        return state

    (fired,) = lax.fori_loop(0, TRIPLE_TRIPS, triple, (jnp.int32(0),))

    for j in range(3):
        pltpu.make_async_copy(tab_hbm.at[:, pl.ds(0, CU)],
                              slabs[j].at[:, pl.ds(0, CU)], sem_slabs[j]).wait()

    def drain(h, carry):
        pltpu.make_async_copy(dst_hbm.at[0], rowbufs.at[pl.ds(0, ROWP)],
                              sem_row).wait()
        return carry

    lax.fori_loop(0, jnp.minimum(fired, 15), drain, 0)


def _streamer_body(user_hbm, item_hbm, tu_hbm, ti_hbm, a_hbm, b_hbm,
                   lu, lc, chunkbuf, slab0, slab1, slab2, rowbufs,
                   sem_s0, sem_s1, sem_s2, sem_row):
    t = lax.axis_index("s") * NC + lax.axis_index("c")
    slabs = [slab0, slab1, slab2]
    sem_slabs = [sem_s0, sem_s1, sem_s2]
    _table_pass(t, user_hbm, a_hbm, tu_hbm, chunkbuf, lu, lc, chunkbuf,
                slabs, rowbufs, sem_slabs, sem_row)
    _table_pass(t, item_hbm, b_hbm, ti_hbm, chunkbuf, lu, lc, chunkbuf,
                slabs, rowbufs, sem_slabs, sem_row)


def _finish_body(a_hbm, b_hbm, w_hbm, out_hbm, rows_a, rows_b, w_v, out_v, sem):
    t = lax.axis_index("s") * NC + lax.axis_index("c")
    base = t * B_PER_W
    ca = pltpu.async_copy(
        a_hbm.at[pl.ds(base, B_PER_W), pl.ds(0, EMBED)], rows_a, sem)
    cb = pltpu.async_copy(
        b_hbm.at[pl.ds(base, B_PER_W), pl.ds(0, EMBED)], rows_b, sem)
    pltpu.sync_copy(w_hbm, w_v)
    ca.wait()
    cb.wait()
    wq = [w_v[pl.ds(q * L, L)] for q in range(EMBED // L)]
    lane = lax.iota(jnp.int32, L)

    def elem(b, carry):
        ur = rows_a.at[b]
        ir = rows_b.at[b]
        s = ur[pl.ds(0, L)] * ir[pl.ds(0, L)] * wq[0]
        for q in range(1, EMBED // L):
            s = s + ur[pl.ds(q * L, L)] * ir[pl.ds(q * L, L)] * wq[q]
        tot = jnp.sum(s)
        plsc.store_scatter(out_v, [jnp.full((L,), b, jnp.int32)],
                           jnp.full((L,), 0.0, jnp.float32) + tot,
                           mask=lane < 1)
        return carry

    lax.fori_loop(0, B_PER_W, elem, 0, unroll=4)

    def group(g, carry):
        v = out_v[pl.ds(g * L, L)]
        out_v[pl.ds(g * L, L)] = 1.0 / (1.0 + jnp.exp(-v))
        return carry

    lax.fori_loop(0, B_PER_W // L, group, 0, unroll=4)
    pltpu.sync_copy(out_v, out_hbm.at[pl.ds(base, B_PER_W)])


def kernel(user, item, user_table, item_table, edge_weight):
    mesh = plsc.VectorSubcoreMesh(
        core_axis_name="c", subcore_axis_name="s", num_cores=NC, num_subcores=NS)
    streamer = functools.partial(
        pl.kernel,
        out_type=(jax.ShapeDtypeStruct((BATCH, ROWP), jnp.float32),
                  jax.ShapeDtypeStruct((BATCH, ROWP), jnp.float32)),
        mesh=mesh,
        compiler_params=pltpu.CompilerParams(
            needs_layout_passes=False, use_tc_tiling_on_sc=True),
        scratch_types=[
            pltpu.VMEM((BATCH + L,), jnp.int32),
            pltpu.VMEM((BATCH + L,), jnp.int32),
            pltpu.VMEM((BATCH + L,), jnp.int32),
            pltpu.VMEM((EMBED, CU + 1), jnp.float32),
            pltpu.VMEM((EMBED, CU + 1), jnp.float32),
            pltpu.VMEM((EMBED, CU + 1), jnp.float32),
            pltpu.VMEM((16 * ROWP,), jnp.float32),
            pltpu.SemaphoreType.DMA,
            pltpu.SemaphoreType.DMA,
            pltpu.SemaphoreType.DMA,
            pltpu.SemaphoreType.DMA,
        ],
    )(_streamer_body)
    finisher = functools.partial(
        pl.kernel,
        out_type=jax.ShapeDtypeStruct((BATCH,), jnp.float32),
        mesh=mesh,
        compiler_params=pltpu.CompilerParams(
            needs_layout_passes=False, use_tc_tiling_on_sc=False),
        scratch_types=[
            pltpu.VMEM((B_PER_W, EMBED), jnp.float32),
            pltpu.VMEM((B_PER_W, EMBED), jnp.float32),
            pltpu.VMEM((EMBED,), jnp.float32),
            pltpu.VMEM((B_PER_W,), jnp.float32),
            pltpu.SemaphoreType.DMA,
        ],
    )(_finish_body)

    a, b = streamer(user.astype(jnp.int32), item.astype(jnp.int32),
                    user_table.T, item_table.T)
    return finisher(a, b, edge_weight.reshape(EMBED))
